# baseline (speedup 1.0000x reference)
import functools

import jax
import jax.numpy as jnp
from jax import lax
from jax.experimental import pallas as pl
from jax.experimental.pallas import tpu as pltpu
from jax.experimental.pallas import tpu_sc as plsc

_NC, _NS, _L = 2, 16, 16
_NW = _NC * _NS
_ROWS = 128
_ROW = 2 * 65536
_ROWS_PER_W = _ROWS // _NW
_ZCHUNK = 16384
_ZITERS = _ZCHUNK // _L


def _body(audio_hbm, src_hbm, out_hbm, src_v, zbuf, sem):
    wid = lax.axis_index("s") * _NC + lax.axis_index("c")

    pltpu.sync_copy(src_hbm, src_v)

    zero = jnp.zeros((_L,), jnp.float32)

    def _zfill(i, carry):
        zbuf[pl.ds(i * _L, _L)] = zero
        return carry

    lax.fori_loop(0, _ZITERS, _zfill, 0)

    for j in range(_ROWS_PER_W):
        r = wid * _ROWS_PER_W + j
        b = r // 8
        s = r % 8
        src_b = plsc.load_gather(src_v, [jnp.full((_L,), b, jnp.int32)])
        is_zero_row = src_b[0] == s

        @pl.when(is_zero_row)
        def _():
            for c in range(_ROW // _ZCHUNK):
                pltpu.sync_copy(zbuf, out_hbm.at[r, pl.ds(c * _ZCHUNK, _ZCHUNK)])

        @pl.when(jnp.logical_not(is_zero_row))
        def _():
            pltpu.sync_copy(audio_hbm.at[r], out_hbm.at[r])


def kernel(audio, source_to_zero):
    batch, streams, channels, time = audio.shape
    flat = audio.reshape(batch * streams, channels * time)
    mesh = plsc.VectorSubcoreMesh(core_axis_name="c", subcore_axis_name="s")
    out = pl.kernel(
        _body,
        out_type=jax.ShapeDtypeStruct(flat.shape, flat.dtype),
        mesh=mesh,
        scratch_types=[
            pltpu.VMEM((batch,), jnp.int32),
            pltpu.VMEM((_ZCHUNK,), jnp.float32),
            pltpu.SemaphoreType.DMA,
        ],
        compiler_params=pltpu.CompilerParams(needs_layout_passes=False),
    )(flat, source_to_zero)
    return out.reshape(audio.shape)


# speedup vs baseline: 11.2647x; 11.2647x over previous
import jax
import jax.numpy as jnp
from jax import lax
from jax.experimental import pallas as pl
from jax.experimental.pallas import tpu as pltpu
from jax.experimental.pallas import tpu_sc as plsc

_NC, _NS, _L = 2, 16, 16
_NW = _NC * _NS
_ROWS = 128
_ROW = 2 * 65536
_ROWS_PER_W = _ROWS // _NW
_STREAMS = 8
_CH = 32768
_CPR = _ROW // _CH
_N = _ROWS_PER_W * _CPR
_ZN = 16384
_ZPC = _CH // _ZN


def _body(audio_hbm, src_hbm, out_hbm,
          src_v, zbuf, buf0, buf1, buf2,
          gsem0, gsem1, gsem2, ssem0, ssem1, ssem2, zsem):
    wid = lax.axis_index("s") * _NC + lax.axis_index("c")
    bufs = (buf0, buf1, buf2)
    gsems = (gsem0, gsem1, gsem2)
    ssems = (ssem0, ssem1, ssem2)

    pltpu.sync_copy(src_hbm, src_v)

    flags = []
    for j in range(_ROWS_PER_W):
        r = wid * _ROWS_PER_W + j
        b = r // _STREAMS
        s = r % _STREAMS
        src_b = plsc.load_gather(src_v, [jnp.full((_L,), b, jnp.int32)])
        flags.append(src_b[0] == s)

    def pred(k):
        return flags[k // _CPR]

    def row(k):
        return wid * _ROWS_PER_W + (k // _CPR)

    def gdesc(k):
        return (audio_hbm.at[row(k), pl.ds((k % _CPR) * _CH, _CH)],
                bufs[k % 3], gsems[k % 3])

    def sdesc(k):
        return (bufs[k % 3],
                out_hbm.at[row(k), pl.ds((k % _CPR) * _CH, _CH)],
                ssems[k % 3])

    for k in (0, 1):

        @pl.when(jnp.logical_not(pred(k)))
        def _(k=k):
            pltpu.async_copy(*gdesc(k))

    zero = jnp.zeros((_L,), jnp.float32)

    def _zfill(i, carry):
        for u in range(16):
            zbuf[pl.ds((i * 16 + u) * _L, _L)] = zero
        return carry

    lax.fori_loop(0, _ZN // (_L * 16), _zfill, 0)

    for k in range(_N):
        m = k % 3
        q = k + 1
        if 2 <= q < _N:
            if q - 3 >= 0:

                @pl.when(jnp.logical_not(pred(q - 3)))
                def _(q=q):
                    pltpu.make_async_copy(*sdesc(q - 3)).wait()

            @pl.when(jnp.logical_not(pred(q)))
            def _(q=q):
                pltpu.async_copy(*gdesc(q))

        @pl.when(pred(k))
        def _(k=k):
            for h in range(_ZPC):
                pltpu.async_copy(
                    zbuf,
                    out_hbm.at[row(k), pl.ds((k % _CPR) * _CH + h * _ZN, _ZN)],
                    zsem)

        @pl.when(jnp.logical_not(pred(k)))
        def _(k=k, m=m):
            pltpu.make_async_copy(*gdesc(k)).wait()
            pltpu.async_copy(*sdesc(k))

    for k in (_N - 3, _N - 2, _N - 1):

        @pl.when(jnp.logical_not(pred(k)))
        def _(k=k):
            pltpu.make_async_copy(*sdesc(k)).wait()

    for k in range(_N):

        @pl.when(pred(k))
        def _(k=k):
            for h in range(_ZPC):
                pltpu.make_async_copy(
                    zbuf,
                    out_hbm.at[row(k), pl.ds((k % _CPR) * _CH + h * _ZN, _ZN)],
                    zsem).wait()


def kernel(audio, source_to_zero):
    batch, streams, channels, time = audio.shape
    flat = audio.reshape(batch * streams, channels * time)
    mesh = plsc.VectorSubcoreMesh(core_axis_name="c", subcore_axis_name="s")
    out = pl.kernel(
        _body,
        out_type=jax.ShapeDtypeStruct(flat.shape, flat.dtype),
        mesh=mesh,
        scratch_types=[
            pltpu.VMEM((batch,), jnp.int32),
            pltpu.VMEM((_ZN,), jnp.float32),
            pltpu.VMEM((_CH,), jnp.float32),
            pltpu.VMEM((_CH,), jnp.float32),
            pltpu.VMEM((_CH,), jnp.float32),
        ] + [pltpu.SemaphoreType.DMA] * 7,
        compiler_params=pltpu.CompilerParams(needs_layout_passes=False),
    )(flat, source_to_zero)
    return out.reshape(audio.shape)
